# baseline (speedup 1.0000x reference)
import functools

import jax
import jax.numpy as jnp
from jax import lax
from jax.experimental import pallas as pl
from jax.experimental.pallas import tpu as pltpu
from jax.experimental.pallas import tpu_sc as plsc

VOCAB = 1000000
EMB = 64
HID = 128
NCLS = 20
B = 4096
L = 200

NCP = 32
PBLK = 32768
PSH = PBLK.bit_length() - 1
KSH = PSH - 2
QMASK = PBLK // 4 - 1
PGRID = (VOCAB + PBLK - 1) // PBLK
PROWS = PGRID * PBLK // 4

NC = 2
NS = 16
NW = NC * NS
SENT_PER_W = B // NW
CH0 = 104
CH1 = L - CH0
IDX_W = SENT_PER_W * L


def _tc_project(tableT, W_h, W_fcp):
    def body(t_ref, wh_ref, wfc_ref, out_ref):
        w = jnp.dot(wh_ref[...], wfc_ref[...],
                    preferred_element_type=jnp.float32)
        p = lax.dot_general(t_ref[...].astype(jnp.bfloat16),
                            w.astype(jnp.bfloat16), (((0,), (0,)), ((), ())),
                            preferred_element_type=jnp.float32)
        q4 = PBLK // 4
        for k in range(4):
            out_ref[:, 32 * k:32 * (k + 1)] = p[q4 * k:q4 * (k + 1), :]

    return pl.pallas_call(
        body,
        grid=(PGRID,),
        in_specs=[
            pl.BlockSpec((EMB, PBLK), lambda i: (0, i)),
            pl.BlockSpec((EMB, HID), lambda i: (0, 0)),
            pl.BlockSpec((HID, NCP), lambda i: (0, 0)),
        ],
        out_specs=pl.BlockSpec((PBLK // 4, 128), lambda i: (i, 0)),
        out_shape=jax.ShapeDtypeStruct((PROWS, 128), jnp.float32),
    )(tableT, W_h, W_fcp)


def _sc_sum_pool(text_flat, ptab):
    mesh = plsc.VectorSubcoreMesh(core_axis_name="c", subcore_axis_name="s")

    @functools.partial(
        pl.kernel,
        mesh=mesh,
        out_type=jax.ShapeDtypeStruct((B, NCP), jnp.float32),
        compiler_params=pltpu.CompilerParams(use_tc_tiling_on_sc=False),
        scratch_types=[
            pltpu.VMEM((IDX_W,), jnp.int32),
            pltpu.VMEM((L, NCP), jnp.float32),
            pltpu.VMEM((L, NCP), jnp.float32),
            pltpu.VMEM((SENT_PER_W, NCP), jnp.float32),
            pltpu.SemaphoreType.DMA,
            pltpu.SemaphoreType.DMA,
        ],
    )
    def k(text_hbm, p_hbm, out_hbm, idx_v, rows_a, rows_b, out_v, s_a, s_b):
        wid = lax.axis_index("s") * NC + lax.axis_index("c")
        pltpu.sync_copy(text_hbm.at[pl.ds(wid * IDX_W, IDX_W)], idx_v)

        def remap(i, carry):
            v = idx_v[pl.ds(i * 16, 16)]
            vp = ((v >> PSH) << PSH) + ((v & QMASK) << 2) + ((v & (PBLK - 1)) >> KSH)
            idx_v[pl.ds(i * 16, 16)] = vp
            return carry

        lax.fori_loop(0, IDX_W // 16, remap, 0)

        def start_gather(s, rows, sem):
            pltpu.async_copy(p_hbm.at[idx_v.at[pl.ds(s * L, CH0)]],
                             rows.at[pl.ds(0, CH0)], sem)
            pltpu.async_copy(p_hbm.at[idx_v.at[pl.ds(s * L + CH0, CH1)]],
                             rows.at[pl.ds(CH0, CH1)], sem)

        def wait_gather(rows, sem):
            pltpu.make_async_copy(p_hbm.at[idx_v.at[pl.ds(0, CH0)]],
                                  rows.at[pl.ds(0, CH0)], sem).wait()
            pltpu.make_async_copy(p_hbm.at[idx_v.at[pl.ds(0, CH1)]],
                                  rows.at[pl.ds(CH0, CH1)], sem).wait()

        z = jnp.zeros((16,), jnp.float32)

        def accumulate(s, rows):
            def oct_(i, acc):
                r = i * 8
                a = list(acc)
                for rr in range(8):
                    for d in range(2):
                        a[(rr % 4) * 2 + d] += rows[r + rr, pl.ds(16 * d, 16)]
                return tuple(a)

            acc = lax.fori_loop(0, L // 8, oct_, (z,) * 8)
            out_v[s, pl.ds(0, 16)] = (acc[0] + acc[2]) + (acc[4] + acc[6])
            out_v[s, pl.ds(16, 16)] = (acc[1] + acc[3]) + (acc[5] + acc[7])

        start_gather(0, rows_a, s_a)
        start_gather(1, rows_b, s_b)

        def pair(j, carry):
            sa = 2 * j
            sb = 2 * j + 1
            wait_gather(rows_a, s_a)
            accumulate(sa, rows_a)

            @pl.when(sa + 2 < SENT_PER_W)
            def _():
                start_gather(sa + 2, rows_a, s_a)

            wait_gather(rows_b, s_b)
            accumulate(sb, rows_b)

            @pl.when(sb + 2 < SENT_PER_W)
            def _():
                start_gather(sb + 2, rows_b, s_b)

            return carry

        lax.fori_loop(0, SENT_PER_W // 2, pair, 0)
        pltpu.sync_copy(out_v, out_hbm.at[pl.ds(wid * SENT_PER_W, SENT_PER_W)])

    return k(text_flat, ptab)


def _tc_epilogue(sums, b_h, W_fcp, b_fcp):
    def body(s_ref, bh_ref, wfc_ref, bfc_ref, out_ref):
        c = jnp.dot(bh_ref[...], wfc_ref[...],
                    preferred_element_type=jnp.float32) + bfc_ref[...]
        o = s_ref[...] * jnp.float32(1.0 / L) + c
        m = jnp.max(o, axis=1, keepdims=True)
        e = jnp.exp(o - m)
        sm = e / jnp.sum(e, axis=1, keepdims=True)
        out_ref[...] = sm[:, :NCLS]

    return pl.pallas_call(
        body,
        out_shape=jax.ShapeDtypeStruct((B, NCLS), jnp.float32),
    )(sums, b_h.reshape(1, HID), W_fcp, b_fcp)


def kernel(text, words_per_sentence, emb_table, W_h, b_h, W_fc, b_fc):
    del words_per_sentence
    text_flat = text.astype(jnp.int32).reshape(B * L)
    W_fcp = jnp.pad(W_fc, ((0, 0), (0, NCP - NCLS)))
    b_fcp = jnp.pad(b_fc, (0, NCP - NCLS),
                    constant_values=-1e30).reshape(1, NCP)
    ppack = _tc_project(emb_table.T, W_h, W_fcp)
    ptab = ppack.reshape(PGRID * PBLK, NCP)
    sums = _sc_sum_pool(text_flat, ptab)
    return _tc_epilogue(sums, b_h, W_fcp, b_fcp)


# speedup vs baseline: 1.1202x; 1.1202x over previous
import functools

import jax
import jax.numpy as jnp
from jax import lax
from jax.experimental import pallas as pl
from jax.experimental.pallas import tpu as pltpu
from jax.experimental.pallas import tpu_sc as plsc

VOCAB = 1000000
EMB = 64
HID = 128
NCLS = 20
B = 4096
L = 200

NCP = 32
PBLK = 32768
PSH = PBLK.bit_length() - 1
KSH = PSH - 2
QMASK = PBLK // 4 - 1
PGRID = (VOCAB + PBLK - 1) // PBLK
PROWS = PGRID * PBLK // 4

NC = 2
NS = 16
NW = NC * NS
SENT_PER_W = B // NW
CH0 = 104
CH1 = L - CH0
IDX_W = SENT_PER_W * L


def _tc_project(tableT, W_h, W_fcp):
    def body(t_ref, wh_ref, wfc_ref, out_ref):
        w = jnp.dot(wh_ref[...], wfc_ref[...],
                    preferred_element_type=jnp.float32)
        p = lax.dot_general(t_ref[...].astype(jnp.bfloat16),
                            w.astype(jnp.bfloat16), (((0,), (0,)), ((), ())),
                            preferred_element_type=jnp.float32)
        q4 = PBLK // 4
        for k in range(4):
            out_ref[:, 32 * k:32 * (k + 1)] = p[q4 * k:q4 * (k + 1), :]

    return pl.pallas_call(
        body,
        grid=(PGRID,),
        in_specs=[
            pl.BlockSpec((EMB, PBLK), lambda i: (0, i)),
            pl.BlockSpec((EMB, HID), lambda i: (0, 0)),
            pl.BlockSpec((HID, NCP), lambda i: (0, 0)),
        ],
        out_specs=pl.BlockSpec((PBLK // 4, 128), lambda i: (i, 0)),
        out_shape=jax.ShapeDtypeStruct((PROWS, 128), jnp.float32),
    )(tableT, W_h, W_fcp)


def _sc_sum_pool(text_flat, ptab):
    mesh = plsc.VectorSubcoreMesh(core_axis_name="c", subcore_axis_name="s")

    @functools.partial(
        pl.kernel,
        mesh=mesh,
        out_type=jax.ShapeDtypeStruct((B, NCP), jnp.float32),
        compiler_params=pltpu.CompilerParams(use_tc_tiling_on_sc=False),
        scratch_types=[
            pltpu.VMEM((IDX_W,), jnp.int32),
            pltpu.VMEM((L, NCP), jnp.float32),
            pltpu.VMEM((L, NCP), jnp.float32),
            pltpu.VMEM((L, NCP), jnp.float32),
            pltpu.VMEM((L, NCP), jnp.float32),
            pltpu.VMEM((SENT_PER_W, NCP), jnp.float32),
            pltpu.SemaphoreType.DMA,
            pltpu.SemaphoreType.DMA,
            pltpu.SemaphoreType.DMA,
            pltpu.SemaphoreType.DMA,
        ],
    )
    def k(text_hbm, p_hbm, out_hbm, idx_v, rows_a, rows_b, rows_c, rows_d,
          out_v, s_a, s_b, s_c, s_d):
        wid = lax.axis_index("s") * NC + lax.axis_index("c")
        pltpu.sync_copy(text_hbm.at[pl.ds(wid * IDX_W, IDX_W)], idx_v)

        def remap(i, carry):
            v = idx_v[pl.ds(i * 16, 16)]
            vp = ((v >> PSH) << PSH) + ((v & QMASK) << 2) + ((v & (PBLK - 1)) >> KSH)
            idx_v[pl.ds(i * 16, 16)] = vp
            return carry

        lax.fori_loop(0, IDX_W // 16, remap, 0)

        def start_gather(s, rows, sem):
            pltpu.async_copy(p_hbm.at[idx_v.at[pl.ds(s * L, CH0)]],
                             rows.at[pl.ds(0, CH0)], sem)
            pltpu.async_copy(p_hbm.at[idx_v.at[pl.ds(s * L + CH0, CH1)]],
                             rows.at[pl.ds(CH0, CH1)], sem)

        def wait_gather(rows, sem):
            pltpu.make_async_copy(p_hbm.at[idx_v.at[pl.ds(0, CH0)]],
                                  rows.at[pl.ds(0, CH0)], sem).wait()
            pltpu.make_async_copy(p_hbm.at[idx_v.at[pl.ds(0, CH1)]],
                                  rows.at[pl.ds(CH0, CH1)], sem).wait()

        z = jnp.zeros((16,), jnp.float32)

        def accumulate(s, rows):
            def oct_(i, acc):
                r = i * 8
                a = list(acc)
                for rr in range(8):
                    for d in range(2):
                        a[(rr % 4) * 2 + d] += rows[r + rr, pl.ds(16 * d, 16)]
                return tuple(a)

            acc = lax.fori_loop(0, L // 8, oct_, (z,) * 8)
            out_v[s, pl.ds(0, 16)] = (acc[0] + acc[2]) + (acc[4] + acc[6])
            out_v[s, pl.ds(16, 16)] = (acc[1] + acc[3]) + (acc[5] + acc[7])

        slots = ((rows_a, s_a), (rows_b, s_b), (rows_c, s_c), (rows_d, s_d))
        for kk in range(4):
            start_gather(kk, slots[kk][0], slots[kk][1])

        def quad_loop(j, carry):
            base = 4 * j
            for kk in range(4):
                rows, sem = slots[kk]
                wait_gather(rows, sem)
                accumulate(base + kk, rows)

                @pl.when(base + kk + 4 < SENT_PER_W)
                def _():
                    start_gather(base + kk + 4, rows, sem)

            return carry

        lax.fori_loop(0, SENT_PER_W // 4, quad_loop, 0)
        pltpu.sync_copy(out_v, out_hbm.at[pl.ds(wid * SENT_PER_W, SENT_PER_W)])

    return k(text_flat, ptab)


def _tc_epilogue(sums, b_h, W_fcp, b_fcp):
    def body(s_ref, bh_ref, wfc_ref, bfc_ref, out_ref):
        c = jnp.dot(bh_ref[...], wfc_ref[...],
                    preferred_element_type=jnp.float32) + bfc_ref[...]
        o = s_ref[...] * jnp.float32(1.0 / L) + c
        m = jnp.max(o, axis=1, keepdims=True)
        e = jnp.exp(o - m)
        sm = e / jnp.sum(e, axis=1, keepdims=True)
        out_ref[...] = sm[:, :NCLS]

    return pl.pallas_call(
        body,
        out_shape=jax.ShapeDtypeStruct((B, NCLS), jnp.float32),
    )(sums, b_h.reshape(1, HID), W_fcp, b_fcp)


def kernel(text, words_per_sentence, emb_table, W_h, b_h, W_fc, b_fc):
    del words_per_sentence
    text_flat = text.astype(jnp.int32).reshape(B * L)
    W_fcp = jnp.pad(W_fc, ((0, 0), (0, NCP - NCLS)))
    b_fcp = jnp.pad(b_fc, (0, NCP - NCLS),
                    constant_values=-1e30).reshape(1, NCP)
    ppack = _tc_project(emb_table.T, W_h, W_fcp)
    ptab = ppack.reshape(PGRID * PBLK, NCP)
    sums = _sc_sum_pool(text_flat, ptab)
    return _tc_epilogue(sums, b_h, W_fcp, b_fcp)
